# baseline (device time: 7781 ns/iter reference)
import jax
import jax.numpy as jnp
from jax import lax
from jax.experimental import pallas as pl
from jax.experimental.pallas import tpu as pltpu


def kernel(x, pi):
    s, m, n = x.shape

    h = m // 2

    def body(pi_ref, x_ref, out_ref, send_buf, recv_buf, send_sems, recv_sems):
        my_x = lax.axis_index("x")
        my_y = lax.axis_index("y")
        my_z = lax.axis_index("z")
        other_x = 1 - my_x

        barrier = pltpu.get_barrier_semaphore()
        pl.semaphore_signal(
            barrier,
            inc=1,
            device_id=(other_x, my_y, my_z),
            device_id_type=pl.DeviceIdType.MESH,
        )
        send_buf[:, :h, :] = x_ref[:, :h, :].astype(jnp.bfloat16)
        pl.semaphore_wait(barrier, 1)

        def chunk_rdma(k):
            rows = pl.ds(k * h, h)
            return pltpu.make_async_remote_copy(
                src_ref=send_buf.at[:, rows, :],
                dst_ref=recv_buf.at[:, rows, :],
                send_sem=send_sems.at[k],
                recv_sem=recv_sems.at[k],
                device_id=(pi_ref[my_x], my_y, my_z),
                device_id_type=pl.DeviceIdType.MESH,
            )

        rdma0 = chunk_rdma(0)
        rdma1 = chunk_rdma(1)
        rdma0.start()
        send_buf[:, h:, :] = x_ref[:, h:, :].astype(jnp.bfloat16)
        rdma1.start()
        rdma0.wait_recv()
        out_ref[:, :h, :] = recv_buf[:, :h, :].astype(jnp.float32)
        rdma1.wait_recv()
        out_ref[:, h:, :] = recv_buf[:, h:, :].astype(jnp.float32)
        rdma0.wait_send()
        rdma1.wait_send()

    return pl.pallas_call(
        body,
        out_shape=jax.ShapeDtypeStruct((s, m, n), jnp.float32),
        in_specs=[
            pl.BlockSpec(memory_space=pltpu.SMEM),
            pl.BlockSpec(memory_space=pltpu.VMEM),
        ],
        out_specs=pl.BlockSpec(memory_space=pltpu.VMEM),
        scratch_shapes=[
            pltpu.VMEM((s, m, n), jnp.bfloat16),
            pltpu.VMEM((s, m, n), jnp.bfloat16),
            pltpu.SemaphoreType.DMA((2,)),
            pltpu.SemaphoreType.DMA((2,)),
        ],
        compiler_params=pltpu.CompilerParams(collective_id=0),
    )(pi, x)


# device time: 7742 ns/iter; 1.0050x vs baseline; 1.0050x over previous
import jax
import jax.numpy as jnp
from jax import lax
from jax.experimental import pallas as pl
from jax.experimental.pallas import tpu as pltpu


def kernel(x, pi):
    s, m, n = x.shape

    def body(pi_ref, x_ref, out_ref, send_buf, recv_buf, send_sem, recv_sem):
        my_x = lax.axis_index("x")
        my_y = lax.axis_index("y")
        my_z = lax.axis_index("z")
        other_x = 1 - my_x

        barrier = pltpu.get_barrier_semaphore()
        pl.semaphore_signal(
            barrier,
            inc=1,
            device_id=(other_x, my_y, my_z),
            device_id_type=pl.DeviceIdType.MESH,
        )
        send_buf[...] = x_ref[...].astype(jnp.bfloat16)
        pl.semaphore_wait(barrier, 1)

        rdma = pltpu.make_async_remote_copy(
            src_ref=send_buf,
            dst_ref=recv_buf,
            send_sem=send_sem,
            recv_sem=recv_sem,
            device_id=(pi_ref[my_x], my_y, my_z),
            device_id_type=pl.DeviceIdType.MESH,
        )
        rdma.start()
        rdma.wait()

        out_ref[...] = recv_buf[...].astype(jnp.float32)

    return pl.pallas_call(
        body,
        out_shape=jax.ShapeDtypeStruct((s, m, n), jnp.float32),
        in_specs=[
            pl.BlockSpec(memory_space=pltpu.SMEM),
            pl.BlockSpec(memory_space=pltpu.VMEM),
        ],
        out_specs=pl.BlockSpec(memory_space=pltpu.VMEM),
        scratch_shapes=[
            pltpu.VMEM((s, m, n), jnp.bfloat16),
            pltpu.VMEM((s, m, n), jnp.bfloat16),
            pltpu.SemaphoreType.DMA,
            pltpu.SemaphoreType.DMA,
        ],
        compiler_params=pltpu.CompilerParams(collective_id=0),
    )(pi, x)
